# baseline (speedup 1.0000x reference)
import jax
import jax.numpy as jnp
import numpy as np
from jax import lax
from jax.experimental import pallas as pl
from jax.experimental.pallas import tpu as pltpu
from jax.experimental.pallas import tpu_sc as plsc

_FACTOR = 8

_rng = np.random.RandomState(0)
_FIXED_IDX = np.stack(
    [_rng.choice(_FACTOR, _FACTOR, replace=False),
     _rng.choice(_FACTOR, _FACTOR, replace=False)],
).astype(np.int32)

_FIXED_PAD = np.zeros((8, 128), np.int32)
_FIXED_PAD[:2, :_FACTOR] = _FIXED_IDX

_NUM_WORKERS = 32


def _sc_copy_body(img_hbm, out_img_hbm):
    c = lax.axis_index("c")
    s = lax.axis_index("s")
    wid = s * 2 + c
    cpw = img_hbm.shape[0] // _NUM_WORKERS
    base = wid * cpw
    pltpu.sync_copy(img_hbm.at[pl.ds(base, cpw)],
                    out_img_hbm.at[pl.ds(base, cpw)])


def _idx_body(idx_ref, fixed_ref, out_idx_ref):
    idx = idx_ref[...]
    nz = jnp.any(idx != 0)
    out_idx_ref[...] = jnp.where(nz, idx, fixed_ref[...])


def kernel(img, indices):
    c, h, w = img.shape

    sc_copy = pl.kernel(
        _sc_copy_body,
        out_type=jax.ShapeDtypeStruct((c, h, w), img.dtype),
        mesh=plsc.VectorSubcoreMesh(core_axis_name="c", subcore_axis_name="s"),
    )
    out_img = sc_copy(img)

    idx_pad = jnp.zeros((8, 128), jnp.int32).at[:2, :_FACTOR].set(indices)
    fixed_pad = jnp.asarray(_FIXED_PAD)
    out_idx_pad = pl.pallas_call(
        _idx_body,
        out_shape=jax.ShapeDtypeStruct((8, 128), jnp.int32),
    )(idx_pad, fixed_pad)

    return out_img, out_idx_pad[:2, :_FACTOR]


# speedup vs baseline: 46.8494x; 46.8494x over previous
import jax
import jax.numpy as jnp
import numpy as np
from jax.experimental import pallas as pl
from jax.experimental.pallas import tpu as pltpu

_FACTOR = 8

_rng = np.random.RandomState(0)
_FIXED_IDX = np.stack(
    [_rng.choice(_FACTOR, _FACTOR, replace=False),
     _rng.choice(_FACTOR, _FACTOR, replace=False)],
).astype(np.int32)

_C_BLOCK = 8


def _body(idx_ref, img_ref, out_img_ref, out_idx_ref):
    out_img_ref[...] = img_ref[...]

    @pl.when(pl.program_id(0) == 0)
    def _():
        nz = idx_ref[0, 0] != 0
        for i in range(2):
            for j in range(_FACTOR):
                if (i, j) != (0, 0):
                    nz = nz | (idx_ref[i, j] != 0)
        for i in range(2):
            for j in range(_FACTOR):
                out_idx_ref[i, j] = jnp.where(
                    nz, idx_ref[i, j], jnp.int32(_FIXED_IDX[i, j]))


def kernel(img, indices):
    c, h, w = img.shape

    return pl.pallas_call(
        _body,
        grid=(c // _C_BLOCK,),
        in_specs=[
            pl.BlockSpec(memory_space=pltpu.SMEM),
            pl.BlockSpec((_C_BLOCK, h, w), lambda i: (i, 0, 0)),
        ],
        out_specs=[
            pl.BlockSpec((_C_BLOCK, h, w), lambda i: (i, 0, 0)),
            pl.BlockSpec(memory_space=pltpu.SMEM),
        ],
        out_shape=[
            jax.ShapeDtypeStruct((c, h, w), img.dtype),
            jax.ShapeDtypeStruct((2, _FACTOR), jnp.int32),
        ],
    )(indices, img)


# speedup vs baseline: 48.8754x; 1.0432x over previous
import jax
import jax.numpy as jnp
import numpy as np
from jax.experimental import pallas as pl
from jax.experimental.pallas import tpu as pltpu

_FACTOR = 8

_rng = np.random.RandomState(0)
_FIXED_IDX = np.stack(
    [_rng.choice(_FACTOR, _FACTOR, replace=False),
     _rng.choice(_FACTOR, _FACTOR, replace=False)],
).astype(np.int32)

_C_BLOCK = 16


def _body(idx_ref, img_ref, out_img_ref, out_idx_ref):
    out_img_ref[...] = img_ref[...]

    @pl.when(pl.program_id(0) == 0)
    def _():
        nz = idx_ref[0, 0] != 0
        for i in range(2):
            for j in range(_FACTOR):
                if (i, j) != (0, 0):
                    nz = nz | (idx_ref[i, j] != 0)
        for i in range(2):
            for j in range(_FACTOR):
                out_idx_ref[i, j] = jnp.where(
                    nz, idx_ref[i, j], jnp.int32(_FIXED_IDX[i, j]))


def kernel(img, indices):
    c, h, w = img.shape

    return pl.pallas_call(
        _body,
        grid=(c // _C_BLOCK,),
        in_specs=[
            pl.BlockSpec(memory_space=pltpu.SMEM),
            pl.BlockSpec((_C_BLOCK, h, w), lambda i: (i, 0, 0)),
        ],
        out_specs=[
            pl.BlockSpec((_C_BLOCK, h, w), lambda i: (i, 0, 0)),
            pl.BlockSpec(memory_space=pltpu.SMEM),
        ],
        out_shape=[
            jax.ShapeDtypeStruct((c, h, w), img.dtype),
            jax.ShapeDtypeStruct((2, _FACTOR), jnp.int32),
        ],
    )(indices, img)


# speedup vs baseline: 49.5259x; 1.0133x over previous
import jax
import jax.numpy as jnp
import numpy as np
from jax.experimental import pallas as pl
from jax.experimental.pallas import tpu as pltpu

_FACTOR = 8

_rng = np.random.RandomState(0)
_FIXED_IDX = np.stack(
    [_rng.choice(_FACTOR, _FACTOR, replace=False),
     _rng.choice(_FACTOR, _FACTOR, replace=False)],
).astype(np.int32)

_C_BLOCK = 24


def _body(idx_ref, img_ref, out_img_ref, out_idx_ref):
    out_img_ref[...] = img_ref[...]

    @pl.when(pl.program_id(0) == 0)
    def _():
        nz = idx_ref[0, 0] != 0
        for i in range(2):
            for j in range(_FACTOR):
                if (i, j) != (0, 0):
                    nz = nz | (idx_ref[i, j] != 0)
        for i in range(2):
            for j in range(_FACTOR):
                out_idx_ref[i, j] = jnp.where(
                    nz, idx_ref[i, j], jnp.int32(_FIXED_IDX[i, j]))


def kernel(img, indices):
    c, h, w = img.shape

    return pl.pallas_call(
        _body,
        grid=(c // _C_BLOCK,),
        in_specs=[
            pl.BlockSpec(memory_space=pltpu.SMEM),
            pl.BlockSpec((_C_BLOCK, h, w), lambda i: (i, 0, 0)),
        ],
        out_specs=[
            pl.BlockSpec((_C_BLOCK, h, w), lambda i: (i, 0, 0)),
            pl.BlockSpec(memory_space=pltpu.SMEM),
        ],
        out_shape=[
            jax.ShapeDtypeStruct((c, h, w), img.dtype),
            jax.ShapeDtypeStruct((2, _FACTOR), jnp.int32),
        ],
    )(indices, img)
